# initial kernel scaffold (unmeasured)
import jax
import jax.numpy as jnp
from jax import lax
from jax.experimental import pallas as pl
from jax.experimental.pallas import tpu as pltpu

N_DEV = 4
N_HOPS = N_DEV - 1


def kernel(x, w_mat):
    m_total, k_per = x.shape
    _, n = w_mat.shape
    m_per = m_total // N_DEV
    nh = n // 2

    def body(x_ref, w_ref, out_ref, commR, commL, amax_comm,
             sendR, recvR, sendL, recvL, sendA, recvA):
        p = lax.axis_index("i")
        right = lax.rem(p + 1, N_DEV)
        left = lax.rem(p + N_DEV - 1, N_DEV)

        barrier = pltpu.get_barrier_semaphore()
        for nbr in (left, right):
            pl.semaphore_signal(
                barrier, inc=1,
                device_id=(nbr,), device_id_type=pl.DeviceIdType.MESH,
            )
        pl.semaphore_wait(barrier, 2)

        def partial(c, lo, hi):
            return jnp.dot(
                x_ref[pl.ds(c * m_per, m_per), :],
                w_ref[:, lo:hi],
                preferred_element_type=jnp.float32,
            )

        commR[0, :, :] = partial(lax.rem(p + N_DEV - 1, N_DEV), 0, nh)
        commL[0, :, :] = partial(lax.rem(p + 1, N_DEV), nh, n)

        for h in range(N_HOPS):
            rdmaR = pltpu.make_async_remote_copy(
                src_ref=commR.at[h],
                dst_ref=commR.at[h + 1],
                send_sem=sendR.at[h],
                recv_sem=recvR.at[h],
                device_id=(right,),
                device_id_type=pl.DeviceIdType.MESH,
            )
            rdmaL = pltpu.make_async_remote_copy(
                src_ref=commL.at[h],
                dst_ref=commL.at[h + 1],
                send_sem=sendL.at[h],
                recv_sem=recvL.at[h],
                device_id=(left,),
                device_id_type=pl.DeviceIdType.MESH,
            )
            rdmaR.start()
            rdmaL.start()

            cR = lax.rem(p + 2 * N_DEV - 2 - h, N_DEV)
            cL = lax.rem(p + 2 + h, N_DEV)
            dR = partial(cR, 0, nh)
            dL = partial(cL, nh, n)

            rdmaR.wait()
            rdmaL.wait()
            if h < N_HOPS - 1:
                commR[h + 1, :, :] = commR[h + 1, :, :] + dR
                commL[h + 1, :, :] = commL[h + 1, :, :] + dL
            else:
                out_ref[:, 0:nh] = commR[h + 1, :, :] + dR
                out_ref[:, nh:n] = commL[h + 1, :, :] + dL

        amax_local = jnp.max(jnp.abs(out_ref[:, :]))
        amax_comm[0, :, :] = jnp.full((8, 128), amax_local, dtype=jnp.float32)
        for h in range(N_HOPS):
            rdmaA = pltpu.make_async_remote_copy(
                src_ref=amax_comm.at[h],
                dst_ref=amax_comm.at[h + 1],
                send_sem=sendA.at[h],
                recv_sem=recvA.at[h],
                device_id=(right,),
                device_id_type=pl.DeviceIdType.MESH,
            )
            rdmaA.start()
            rdmaA.wait()
        amax_g = amax_comm[0, 0, 0]
        for s in range(1, N_DEV):
            amax_g = jnp.maximum(amax_g, amax_comm[s, 0, 0])

        scale = amax_g / 448.0
        q = jnp.clip(out_ref[:, :] / scale, -448.0, 448.0)
        snapped = q.astype(jnp.float8_e4m3fn).astype(jnp.float32)
        out_ref[:, :] = snapped * scale

    return pl.pallas_call(
        body,
        out_shape=jax.ShapeDtypeStruct((m_per, n), jnp.float32),
        in_specs=[
            pl.BlockSpec(memory_space=pltpu.VMEM),
            pl.BlockSpec(memory_space=pltpu.VMEM),
        ],
        out_specs=pl.BlockSpec(memory_space=pltpu.VMEM),
        scratch_shapes=[
            pltpu.VMEM((N_DEV, m_per, nh), jnp.float32),
            pltpu.VMEM((N_DEV, m_per, n - nh), jnp.float32),
            pltpu.VMEM((N_DEV, 8, 128), jnp.float32),
            pltpu.SemaphoreType.DMA((N_HOPS,)),
            pltpu.SemaphoreType.DMA((N_HOPS,)),
            pltpu.SemaphoreType.DMA((N_HOPS,)),
            pltpu.SemaphoreType.DMA((N_HOPS,)),
            pltpu.SemaphoreType.DMA((N_HOPS,)),
            pltpu.SemaphoreType.DMA((N_HOPS,)),
        ],
        compiler_params=pltpu.CompilerParams(collective_id=0),
    )(x, w_mat)


# baseline (device time: 175869 ns/iter reference)
import jax
import jax.numpy as jnp
from jax import lax
from jax.experimental import pallas as pl
from jax.experimental.pallas import tpu as pltpu

N_DEV = 4
N_HOPS = N_DEV - 1


def kernel(x, w_mat):
    m_total, k_per = x.shape
    _, n = w_mat.shape
    m_per = m_total // N_DEV
    nh = n // 2

    def body(x_ref, w_ref, out_ref, commR, commL, amax_comm,
             sendR, recvR, sendL, recvL, sendA, recvA, creditR, creditL):
        p = lax.axis_index("i")
        right = lax.rem(p + 1, N_DEV)
        left = lax.rem(p + N_DEV - 1, N_DEV)

        barrier = pltpu.get_barrier_semaphore()
        for nbr in (left, right):
            pl.semaphore_signal(
                barrier, inc=1,
                device_id=(nbr,), device_id_type=pl.DeviceIdType.MESH,
            )
        pl.semaphore_wait(barrier, 2)

        def partial(c, lo, hi):
            return jnp.dot(
                x_ref[pl.ds(c * m_per, m_per), :],
                w_ref[:, lo:hi],
                preferred_element_type=jnp.float32,
            )

        def rcopy(src, dst, h):
            return pltpu.make_async_remote_copy(
                src_ref=src, dst_ref=dst,
                send_sem=sendR.at[h], recv_sem=recvR.at[h],
                device_id=(right,), device_id_type=pl.DeviceIdType.MESH,
            )

        def lcopy(src, dst, h):
            return pltpu.make_async_remote_copy(
                src_ref=src, dst_ref=dst,
                send_sem=sendL.at[h], recv_sem=recvL.at[h],
                device_id=(left,), device_id_type=pl.DeviceIdType.MESH,
            )

        commR[0, :, :] = partial(lax.rem(p + 3, N_DEV), 0, nh)
        commL[0, :, :] = partial(lax.rem(p + 1, N_DEV), nh, n)

        r0 = rcopy(commR.at[0], commR.at[1], 0)
        l0 = lcopy(commL.at[0], commL.at[1], 0)
        r0.start()
        l0.start()
        dR = partial(lax.rem(p + 2, N_DEV), 0, nh)
        dL = partial(lax.rem(p + 2, N_DEV), nh, n)
        r0.wait_send()
        pl.semaphore_signal(
            creditR, inc=1, device_id=(left,),
            device_id_type=pl.DeviceIdType.MESH,
        )
        l0.wait_send()
        pl.semaphore_signal(
            creditL, inc=1, device_id=(right,),
            device_id_type=pl.DeviceIdType.MESH,
        )
        r0.wait_recv()
        commR[1, :, :] = commR[1, :, :] + dR
        l0.wait_recv()
        commL[1, :, :] = commL[1, :, :] + dL

        pl.semaphore_wait(creditR, 1)
        r1 = rcopy(commR.at[1], commR.at[0], 1)
        r1.start()
        pl.semaphore_wait(creditL, 1)
        l1 = lcopy(commL.at[1], commL.at[0], 1)
        l1.start()
        dR = partial(lax.rem(p + 1, N_DEV), 0, nh)
        dL = partial(lax.rem(p + 3, N_DEV), nh, n)
        r1.wait_recv()
        commR[0, :, :] = commR[0, :, :] + dR
        l1.wait_recv()
        commL[0, :, :] = commL[0, :, :] + dL
        r1.wait_send()
        l1.wait_send()

        r2 = rcopy(commR.at[0], out_ref.at[:, pl.ds(0, nh)], 2)
        l2 = lcopy(commL.at[0], out_ref.at[:, pl.ds(nh, n - nh)], 2)
        r2.start()
        l2.start()
        dR = partial(p, 0, nh)
        dL = partial(p, nh, n)
        r2.wait_recv()
        out_ref[:, 0:nh] = out_ref[:, 0:nh] + dR
        l2.wait_recv()
        out_ref[:, nh:n] = out_ref[:, nh:n] + dL
        r2.wait_send()
        l2.wait_send()

        amax_local = jnp.max(jnp.abs(out_ref[:, :]))
        amax_comm[0, :, :] = jnp.full((8, 128), amax_local, dtype=jnp.float32)
        for h in range(N_HOPS):
            rdmaA = pltpu.make_async_remote_copy(
                src_ref=amax_comm.at[h],
                dst_ref=amax_comm.at[h + 1],
                send_sem=sendA.at[h],
                recv_sem=recvA.at[h],
                device_id=(right,),
                device_id_type=pl.DeviceIdType.MESH,
            )
            rdmaA.start()
            rdmaA.wait()
        amax_g = amax_comm[0, 0, 0]
        for s in range(1, N_DEV):
            amax_g = jnp.maximum(amax_g, amax_comm[s, 0, 0])

        scale = amax_g / 448.0
        q = jnp.clip(out_ref[:, :] / scale, -448.0, 448.0)
        snapped = q.astype(jnp.float8_e4m3fn).astype(jnp.float32)
        out_ref[:, :] = snapped * scale

    return pl.pallas_call(
        body,
        out_shape=jax.ShapeDtypeStruct((m_per, n), jnp.float32),
        in_specs=[
            pl.BlockSpec(memory_space=pltpu.VMEM),
            pl.BlockSpec(memory_space=pltpu.VMEM),
        ],
        out_specs=pl.BlockSpec(memory_space=pltpu.VMEM),
        scratch_shapes=[
            pltpu.VMEM((2, m_per, nh), jnp.float32),
            pltpu.VMEM((2, m_per, n - nh), jnp.float32),
            pltpu.VMEM((N_DEV, 8, 128), jnp.float32),
            pltpu.SemaphoreType.DMA((N_HOPS,)),
            pltpu.SemaphoreType.DMA((N_HOPS,)),
            pltpu.SemaphoreType.DMA((N_HOPS,)),
            pltpu.SemaphoreType.DMA((N_HOPS,)),
            pltpu.SemaphoreType.DMA((N_HOPS,)),
            pltpu.SemaphoreType.DMA((N_HOPS,)),
            pltpu.SemaphoreType.REGULAR,
            pltpu.SemaphoreType.REGULAR,
        ],
        compiler_params=pltpu.CompilerParams(
            collective_id=0,
            vmem_limit_bytes=100 * 1024 * 1024,
        ),
    )(x, w_mat)


# device time: 166311 ns/iter; 1.0575x vs baseline; 1.0575x over previous
import jax
import jax.numpy as jnp
from jax import lax
from jax.experimental import pallas as pl
from jax.experimental.pallas import tpu as pltpu

N_DEV = 4
N_HOPS = N_DEV - 1
NSUB = 2


def kernel(x, w_mat):
    m_total, k_per = x.shape
    _, n = w_mat.shape
    m_per = m_total // N_DEV
    nh = n // 2
    sw = nh // NSUB

    def body(x_ref, w_ref, out_ref, commR, commL, amaxB,
             sendR, recvR, sendL, recvL, sendA, recvA, creditR, creditL):
        p = lax.axis_index("i")
        right = lax.rem(p + 1, N_DEV)
        left = lax.rem(p + N_DEV - 1, N_DEV)

        barrier = pltpu.get_barrier_semaphore()
        for nbr in (left, right):
            pl.semaphore_signal(
                barrier, inc=1,
                device_id=(nbr,), device_id_type=pl.DeviceIdType.MESH,
            )
        pl.semaphore_wait(barrier, 2)

        def partial(c, lo):
            return jnp.dot(
                x_ref[pl.ds(c * m_per, m_per), :],
                w_ref[:, lo:lo + sw],
                preferred_element_type=jnp.float32,
            )

        def rsub(slot, s):
            return commR.at[slot, :, pl.ds(s * sw, sw)]

        def lsub(slot, s):
            return commL.at[slot, :, pl.ds(s * sw, sw)]

        def mk(src, dst, send_sem, recv_sem, dev):
            return pltpu.make_async_remote_copy(
                src_ref=src, dst_ref=dst, send_sem=send_sem,
                recv_sem=recv_sem, device_id=(dev,),
                device_id_type=pl.DeviceIdType.MESH,
            )

        def hop_sends(h, src_slot, dst_slot):
            def make(s):
                if h == N_HOPS - 1:
                    dstR = out_ref.at[:, pl.ds(s * sw, sw)]
                    dstL = out_ref.at[:, pl.ds(nh + s * sw, sw)]
                else:
                    dstR = rsub(dst_slot, s)
                    dstL = lsub(dst_slot, s)
                r = mk(rsub(src_slot, s), dstR,
                       sendR.at[h, s], recvR.at[h, s], right)
                l = mk(lsub(src_slot, s), dstL,
                       sendL.at[h, s], recvL.at[h, s], left)
                return r, l
            return make

        c_sR = lax.rem(p + 3, N_DEV)
        c_sL = lax.rem(p + 1, N_DEV)
        mk0 = hop_sends(0, 0, 1)
        R = [None] * NSUB
        L = [None] * NSUB
        for s in range(NSUB):
            commR[0, :, s * sw:(s + 1) * sw] = partial(c_sR, s * sw)
            commL[0, :, s * sw:(s + 1) * sw] = partial(c_sL, nh + s * sw)
            R[s], L[s] = mk0(s)
            R[s].start()
            L[s].start()

        arrivals = [
            (lax.rem(p + 2, N_DEV), lax.rem(p + 2, N_DEV)),
            (lax.rem(p + 1, N_DEV), lax.rem(p + 3, N_DEV)),
            (p, p),
        ]

        amax = jnp.float32(0.0)
        for h in range(N_HOPS):
            cR, cL = arrivals[h]
            dR = [partial(cR, s * sw) for s in range(NSUB)]
            dL = [partial(cL, nh + s * sw) for s in range(NSUB)]

            recv_slot = (h + 1) % 2
            last = h == N_HOPS - 1
            mk_next = None if last else hop_sends(h + 1, recv_slot, h % 2)
            Rn = [None] * NSUB
            Ln = [None] * NSUB
            for s in range(NSUB):
                R[s].wait_recv()
                L[s].wait_recv()
                if last:
                    yR = out_ref[:, s * sw:(s + 1) * sw] + dR[s]
                    yL = out_ref[:, nh + s * sw:nh + (s + 1) * sw] + dL[s]
                    out_ref[:, s * sw:(s + 1) * sw] = yR
                    out_ref[:, nh + s * sw:nh + (s + 1) * sw] = yL
                    amax = jnp.maximum(
                        amax,
                        jnp.maximum(jnp.max(jnp.abs(yR)), jnp.max(jnp.abs(yL))),
                    )
                    R[s].wait_send()
                    L[s].wait_send()
                else:
                    commR[recv_slot, :, s * sw:(s + 1) * sw] = (
                        commR[recv_slot, :, s * sw:(s + 1) * sw] + dR[s]
                    )
                    commL[recv_slot, :, s * sw:(s + 1) * sw] = (
                        commL[recv_slot, :, s * sw:(s + 1) * sw] + dL[s]
                    )
                    R[s].wait_send()
                    L[s].wait_send()
                    if h == 0:
                        pl.semaphore_signal(
                            creditR, inc=1, device_id=(left,),
                            device_id_type=pl.DeviceIdType.MESH,
                        )
                        pl.semaphore_signal(
                            creditL, inc=1, device_id=(right,),
                            device_id_type=pl.DeviceIdType.MESH,
                        )
                        pl.semaphore_wait(creditR, 1)
                        pl.semaphore_wait(creditL, 1)
                    Rn[s], Ln[s] = mk_next(s)
                    Rn[s].start()
                    Ln[s].start()
            R, L = Rn, Ln

        amaxB[0, :, :] = jnp.full((8, 128), amax, dtype=jnp.float32)
        e1r = mk(amaxB.at[0], amaxB.at[1], sendA.at[0], recvA.at[0], right)
        e1l = mk(amaxB.at[0], amaxB.at[2], sendA.at[1], recvA.at[1], left)
        e1r.start()
        e1l.start()
        e1r.wait()
        e1l.wait()
        e2r = mk(amaxB.at[1], amaxB.at[3], sendA.at[2], recvA.at[2], right)
        e2r.start()
        e2r.wait()
        amax_g = amaxB[0, 0, 0]
        for slot in range(1, 4):
            amax_g = jnp.maximum(amax_g, amaxB[slot, 0, 0])

        scale = amax_g / 448.0
        q = jnp.clip(out_ref[:, :] / scale, -448.0, 448.0)
        snapped = q.astype(jnp.float8_e4m3fn).astype(jnp.float32)
        out_ref[:, :] = snapped * scale

    return pl.pallas_call(
        body,
        out_shape=jax.ShapeDtypeStruct((m_per, n), jnp.float32),
        in_specs=[
            pl.BlockSpec(memory_space=pltpu.VMEM),
            pl.BlockSpec(memory_space=pltpu.VMEM),
        ],
        out_specs=pl.BlockSpec(memory_space=pltpu.VMEM),
        scratch_shapes=[
            pltpu.VMEM((2, m_per, nh), jnp.float32),
            pltpu.VMEM((2, m_per, n - nh), jnp.float32),
            pltpu.VMEM((4, 8, 128), jnp.float32),
            pltpu.SemaphoreType.DMA((N_HOPS, NSUB)),
            pltpu.SemaphoreType.DMA((N_HOPS, NSUB)),
            pltpu.SemaphoreType.DMA((N_HOPS, NSUB)),
            pltpu.SemaphoreType.DMA((N_HOPS, NSUB)),
            pltpu.SemaphoreType.DMA((3,)),
            pltpu.SemaphoreType.DMA((3,)),
            pltpu.SemaphoreType.REGULAR,
            pltpu.SemaphoreType.REGULAR,
        ],
        compiler_params=pltpu.CompilerParams(
            collective_id=0,
            vmem_limit_bytes=100 * 1024 * 1024,
        ),
    )(x, w_mat)


# device time: 160284 ns/iter; 1.0972x vs baseline; 1.0376x over previous
import jax
import jax.numpy as jnp
from jax import lax
from jax.experimental import pallas as pl
from jax.experimental.pallas import tpu as pltpu

N_DEV = 4
N_HOPS = N_DEV - 1
NSUB = 2


def kernel(x, w_mat):
    m_total, k_per = x.shape
    _, n = w_mat.shape
    m_per = m_total // N_DEV
    nh = n // 2
    sw = nh // NSUB

    def body(x_ref, w_ref, out_ref, xv, wv, commR, commL, amaxB,
             xsem, wsem, sendR, recvR, sendL, recvL, sendA, recvA,
             creditR, creditL):
        p = lax.axis_index("i")
        right = lax.rem(p + 1, N_DEV)
        left = lax.rem(p + N_DEV - 1, N_DEV)

        chunk_of_slot = [
            lax.rem(p + 3, N_DEV), lax.rem(p + 1, N_DEV),
            lax.rem(p + 2, N_DEV), p,
        ]
        xc = [
            pltpu.make_async_copy(
                x_ref.at[pl.ds(chunk_of_slot[k] * m_per, m_per), :],
                xv.at[k], xsem.at[k],
            )
            for k in range(4)
        ]
        wc = [
            pltpu.make_async_copy(
                w_ref.at[:, pl.ds(q * sw, sw)],
                wv.at[:, pl.ds(q * sw, sw)], wsem.at[q],
            )
            for q in range(4)
        ]
        for cp in (wc[0], xc[0], wc[2], xc[1], wc[1], wc[3], xc[2], xc[3]):
            cp.start()

        barrier = pltpu.get_barrier_semaphore()
        for nbr in (left, right):
            pl.semaphore_signal(
                barrier, inc=1,
                device_id=(nbr,), device_id_type=pl.DeviceIdType.MESH,
            )
        pl.semaphore_wait(barrier, 2)

        def partial(slot, lo):
            return jnp.dot(
                xv[slot], wv[:, lo:lo + sw],
                preferred_element_type=jnp.float32,
            )

        def rsub(slot, s):
            return commR.at[slot, :, pl.ds(s * sw, sw)]

        def lsub(slot, s):
            return commL.at[slot, :, pl.ds(s * sw, sw)]

        def mk(src, dst, send_sem, recv_sem, dev):
            return pltpu.make_async_remote_copy(
                src_ref=src, dst_ref=dst, send_sem=send_sem,
                recv_sem=recv_sem, device_id=(dev,),
                device_id_type=pl.DeviceIdType.MESH,
            )

        def hop_sends(h, src_slot, dst_slot):
            def make(s):
                if h == N_HOPS - 1:
                    dstR = out_ref.at[:, pl.ds(s * sw, sw)]
                    dstL = out_ref.at[:, pl.ds(nh + s * sw, sw)]
                else:
                    dstR = rsub(dst_slot, s)
                    dstL = lsub(dst_slot, s)
                r = mk(rsub(src_slot, s), dstR,
                       sendR.at[h, s], recvR.at[h, s], right)
                l = mk(lsub(src_slot, s), dstL,
                       sendL.at[h, s], recvL.at[h, s], left)
                return r, l
            return make

        mk0 = hop_sends(0, 0, 1)
        R = [None] * NSUB
        L = [None] * NSUB
        for s in range(NSUB):
            wc[s].wait()
            if s == 0:
                xc[0].wait()
            commR[0, :, s * sw:(s + 1) * sw] = partial(0, s * sw)
            wc[2 + s].wait()
            if s == 0:
                xc[1].wait()
            commL[0, :, s * sw:(s + 1) * sw] = partial(1, nh + s * sw)
            R[s], L[s] = mk0(s)
            R[s].start()
            L[s].start()

        arrivals = [(2, 2), (1, 0), (3, 3)]

        amax = jnp.float32(0.0)
        for h in range(N_HOPS):
            sR, sL = arrivals[h]
            if h == 0:
                xc[2].wait()
            if h == N_HOPS - 1:
                xc[3].wait()
            dR = [partial(sR, s * sw) for s in range(NSUB)]
            dL = [partial(sL, nh + s * sw) for s in range(NSUB)]

            recv_slot = (h + 1) % 2
            last = h == N_HOPS - 1
            mk_next = None if last else hop_sends(h + 1, recv_slot, h % 2)
            Rn = [None] * NSUB
            Ln = [None] * NSUB
            for s in range(NSUB):
                R[s].wait_recv()
                L[s].wait_recv()
                if last:
                    yR = out_ref[:, s * sw:(s + 1) * sw] + dR[s]
                    yL = out_ref[:, nh + s * sw:nh + (s + 1) * sw] + dL[s]
                    out_ref[:, s * sw:(s + 1) * sw] = yR
                    out_ref[:, nh + s * sw:nh + (s + 1) * sw] = yL
                    amax = jnp.maximum(
                        amax,
                        jnp.maximum(jnp.max(jnp.abs(yR)), jnp.max(jnp.abs(yL))),
                    )
                    R[s].wait_send()
                    L[s].wait_send()
                else:
                    commR[recv_slot, :, s * sw:(s + 1) * sw] = (
                        commR[recv_slot, :, s * sw:(s + 1) * sw] + dR[s]
                    )
                    commL[recv_slot, :, s * sw:(s + 1) * sw] = (
                        commL[recv_slot, :, s * sw:(s + 1) * sw] + dL[s]
                    )
                    R[s].wait_send()
                    L[s].wait_send()
                    if h == 0:
                        pl.semaphore_signal(
                            creditR, inc=1, device_id=(left,),
                            device_id_type=pl.DeviceIdType.MESH,
                        )
                        pl.semaphore_signal(
                            creditL, inc=1, device_id=(right,),
                            device_id_type=pl.DeviceIdType.MESH,
                        )
                        pl.semaphore_wait(creditR, 1)
                        pl.semaphore_wait(creditL, 1)
                    Rn[s], Ln[s] = mk_next(s)
                    Rn[s].start()
                    Ln[s].start()
            R, L = Rn, Ln

        amaxB[0, :, :] = jnp.full((8, 128), amax, dtype=jnp.float32)
        diag = lax.rem(p + 2, N_DEV)
        ex = [
            mk(amaxB.at[0], amaxB.at[1], sendA.at[0], recvA.at[0], right),
            mk(amaxB.at[0], amaxB.at[2], sendA.at[1], recvA.at[1], left),
            mk(amaxB.at[0], amaxB.at[3], sendA.at[2], recvA.at[2], diag),
        ]
        for e in ex:
            e.start()
        for e in ex:
            e.wait()
        amax_g = amaxB[0, 0, 0]
        for slot in range(1, 4):
            amax_g = jnp.maximum(amax_g, amaxB[slot, 0, 0])

        scale = amax_g / 448.0
        q = jnp.clip(out_ref[:, :] / scale, -448.0, 448.0)
        snapped = q.astype(jnp.float8_e4m3fn).astype(jnp.float32)
        out_ref[:, :] = snapped * scale

    return pl.pallas_call(
        body,
        out_shape=jax.ShapeDtypeStruct((m_per, n), jnp.float32),
        in_specs=[
            pl.BlockSpec(memory_space=pltpu.MemorySpace.HBM),
            pl.BlockSpec(memory_space=pltpu.MemorySpace.HBM),
        ],
        out_specs=pl.BlockSpec(memory_space=pltpu.VMEM),
        scratch_shapes=[
            pltpu.VMEM((4, m_per, k_per), jnp.float32),
            pltpu.VMEM((k_per, n), jnp.float32),
            pltpu.VMEM((2, m_per, nh), jnp.float32),
            pltpu.VMEM((2, m_per, n - nh), jnp.float32),
            pltpu.VMEM((4, 8, 128), jnp.float32),
            pltpu.SemaphoreType.DMA((4,)),
            pltpu.SemaphoreType.DMA((4,)),
            pltpu.SemaphoreType.DMA((N_HOPS, NSUB)),
            pltpu.SemaphoreType.DMA((N_HOPS, NSUB)),
            pltpu.SemaphoreType.DMA((N_HOPS, NSUB)),
            pltpu.SemaphoreType.DMA((N_HOPS, NSUB)),
            pltpu.SemaphoreType.DMA((3,)),
            pltpu.SemaphoreType.DMA((3,)),
            pltpu.SemaphoreType.REGULAR,
            pltpu.SemaphoreType.REGULAR,
        ],
        compiler_params=pltpu.CompilerParams(
            collective_id=0,
            vmem_limit_bytes=100 * 1024 * 1024,
        ),
    )(x, w_mat)


# device time: 159488 ns/iter; 1.1027x vs baseline; 1.0050x over previous
import jax
import jax.numpy as jnp
from jax import lax
from jax.experimental import pallas as pl
from jax.experimental.pallas import tpu as pltpu

N_DEV = 4
N_HOPS = N_DEV - 1
NSUB = 4


def kernel(x, w_mat):
    m_total, k_per = x.shape
    _, n = w_mat.shape
    m_per = m_total // N_DEV
    nh = n // 2
    sw = nh // NSUB

    def body(x_ref, w_ref, out_ref, xv, wv, yv, commR, commL, amaxB,
             xsem, wsem, osem, sendR, recvR, sendL, recvL, sendA, recvA,
             creditR, creditL):
        p = lax.axis_index("i")
        right = lax.rem(p + 1, N_DEV)
        left = lax.rem(p + N_DEV - 1, N_DEV)

        wq = n // 4
        chunk_of_slot = [
            lax.rem(p + 3, N_DEV), lax.rem(p + 1, N_DEV),
            lax.rem(p + 2, N_DEV), p,
        ]
        xc = [
            pltpu.make_async_copy(
                x_ref.at[pl.ds(chunk_of_slot[k] * m_per, m_per), :],
                xv.at[k], xsem.at[k],
            )
            for k in range(4)
        ]
        wc = [
            pltpu.make_async_copy(
                w_ref.at[:, pl.ds(q * wq, wq)],
                wv.at[:, pl.ds(q * wq, wq)], wsem.at[q],
            )
            for q in range(4)
        ]
        for cp in (wc[0], xc[0], wc[2], xc[1], wc[1], wc[3], xc[2], xc[3]):
            cp.start()
        waited = set()

        def ensure(cp, key):
            if key not in waited:
                cp.wait()
                waited.add(key)

        barrier = pltpu.get_barrier_semaphore()
        for nbr in (left, right):
            pl.semaphore_signal(
                barrier, inc=1,
                device_id=(nbr,), device_id_type=pl.DeviceIdType.MESH,
            )
        pl.semaphore_wait(barrier, 2)

        def partial(slot, lo):
            return jnp.dot(
                xv[slot], wv[:, lo:lo + sw],
                preferred_element_type=jnp.float32,
            )

        def rsub(slot, s):
            return commR.at[slot, :, pl.ds(s * sw, sw)]

        def lsub(slot, s):
            return commL.at[slot, :, pl.ds(s * sw, sw)]

        def mk(src, dst, send_sem, recv_sem, dev):
            return pltpu.make_async_remote_copy(
                src_ref=src, dst_ref=dst, send_sem=send_sem,
                recv_sem=recv_sem, device_id=(dev,),
                device_id_type=pl.DeviceIdType.MESH,
            )

        def hop_sends(h, src_slot, dst_slot):
            def make(s):
                if h == N_HOPS - 1:
                    dstR = yv.at[:, pl.ds(s * sw, sw)]
                    dstL = yv.at[:, pl.ds(nh + s * sw, sw)]
                else:
                    dstR = rsub(dst_slot, s)
                    dstL = lsub(dst_slot, s)
                r = mk(rsub(src_slot, s), dstR,
                       sendR.at[h, s], recvR.at[h, s], right)
                l = mk(lsub(src_slot, s), dstL,
                       sendL.at[h, s], recvL.at[h, s], left)
                return r, l
            return make

        mk0 = hop_sends(0, 0, 1)
        R = [None] * NSUB
        L = [None] * NSUB
        for s in range(NSUB):
            ensure(wc[(s * sw) // wq], ("w", (s * sw) // wq))
            ensure(xc[0], ("x", 0))
            commR[0, :, s * sw:(s + 1) * sw] = partial(0, s * sw)
            ensure(wc[(nh + s * sw) // wq], ("w", (nh + s * sw) // wq))
            ensure(xc[1], ("x", 1))
            commL[0, :, s * sw:(s + 1) * sw] = partial(1, nh + s * sw)
            R[s], L[s] = mk0(s)
            R[s].start()
            L[s].start()

        arrivals = [(2, 2), (1, 0), (3, 3)]

        amax = jnp.float32(0.0)
        for h in range(N_HOPS):
            sR, sL = arrivals[h]
            if h == 0:
                ensure(xc[2], ("x", 2))
            if h == N_HOPS - 1:
                ensure(xc[3], ("x", 3))
            dR = [partial(sR, s * sw) for s in range(NSUB)]
            dL = [partial(sL, nh + s * sw) for s in range(NSUB)]

            recv_slot = (h + 1) % 2
            last = h == N_HOPS - 1
            mk_next = None if last else hop_sends(h + 1, recv_slot, h % 2)
            Rn = [None] * NSUB
            Ln = [None] * NSUB
            for s in range(NSUB):
                R[s].wait_recv()
                L[s].wait_recv()
                if last:
                    yR = yv[:, s * sw:(s + 1) * sw] + dR[s]
                    yL = yv[:, nh + s * sw:nh + (s + 1) * sw] + dL[s]
                    yv[:, s * sw:(s + 1) * sw] = yR
                    yv[:, nh + s * sw:nh + (s + 1) * sw] = yL
                    amax = jnp.maximum(
                        amax,
                        jnp.maximum(jnp.max(jnp.abs(yR)), jnp.max(jnp.abs(yL))),
                    )
                    R[s].wait_send()
                    L[s].wait_send()
                else:
                    commR[recv_slot, :, s * sw:(s + 1) * sw] = (
                        commR[recv_slot, :, s * sw:(s + 1) * sw] + dR[s]
                    )
                    commL[recv_slot, :, s * sw:(s + 1) * sw] = (
                        commL[recv_slot, :, s * sw:(s + 1) * sw] + dL[s]
                    )
                    R[s].wait_send()
                    L[s].wait_send()
                    if h == 0:
                        pl.semaphore_signal(
                            creditR, inc=1, device_id=(left,),
                            device_id_type=pl.DeviceIdType.MESH,
                        )
                        pl.semaphore_signal(
                            creditL, inc=1, device_id=(right,),
                            device_id_type=pl.DeviceIdType.MESH,
                        )
                        pl.semaphore_wait(creditR, 1)
                        pl.semaphore_wait(creditL, 1)
                    Rn[s], Ln[s] = mk_next(s)
                    Rn[s].start()
                    Ln[s].start()
            R, L = Rn, Ln

        amaxB[0, :, :] = jnp.full((8, 128), amax, dtype=jnp.float32)
        diag = lax.rem(p + 2, N_DEV)
        ex = [
            mk(amaxB.at[0], amaxB.at[1], sendA.at[0], recvA.at[0], right),
            mk(amaxB.at[0], amaxB.at[2], sendA.at[1], recvA.at[1], left),
            mk(amaxB.at[0], amaxB.at[3], sendA.at[2], recvA.at[2], diag),
        ]
        for e in ex:
            e.start()
        for e in ex:
            e.wait()
        amax_g = amaxB[0, 0, 0]
        for slot in range(1, 4):
            amax_g = jnp.maximum(amax_g, amaxB[slot, 0, 0])

        scale = amax_g / 448.0
        n_strips = 4
        stw = n // n_strips
        outcp = []
        for k in range(n_strips):
            q = jnp.clip(yv[:, k * stw:(k + 1) * stw] / scale, -448.0, 448.0)
            snapped = q.astype(jnp.float8_e4m3fn).astype(jnp.float32)
            yv[:, k * stw:(k + 1) * stw] = snapped * scale
            cp = pltpu.make_async_copy(
                yv.at[:, pl.ds(k * stw, stw)],
                out_ref.at[:, pl.ds(k * stw, stw)],
                osem.at[k],
            )
            cp.start()
            outcp.append(cp)
        for cp in outcp:
            cp.wait()

    return pl.pallas_call(
        body,
        out_shape=jax.ShapeDtypeStruct((m_per, n), jnp.float32),
        in_specs=[
            pl.BlockSpec(memory_space=pltpu.MemorySpace.HBM),
            pl.BlockSpec(memory_space=pltpu.MemorySpace.HBM),
        ],
        out_specs=pl.BlockSpec(memory_space=pltpu.MemorySpace.HBM),
        scratch_shapes=[
            pltpu.VMEM((4, m_per, k_per), jnp.float32),
            pltpu.VMEM((k_per, n), jnp.float32),
            pltpu.VMEM((m_per, n), jnp.float32),
            pltpu.VMEM((2, m_per, nh), jnp.float32),
            pltpu.VMEM((2, m_per, n - nh), jnp.float32),
            pltpu.VMEM((4, 8, 128), jnp.float32),
            pltpu.SemaphoreType.DMA((4,)),
            pltpu.SemaphoreType.DMA((4,)),
            pltpu.SemaphoreType.DMA((4,)),
            pltpu.SemaphoreType.DMA((N_HOPS, NSUB)),
            pltpu.SemaphoreType.DMA((N_HOPS, NSUB)),
            pltpu.SemaphoreType.DMA((N_HOPS, NSUB)),
            pltpu.SemaphoreType.DMA((N_HOPS, NSUB)),
            pltpu.SemaphoreType.DMA((3,)),
            pltpu.SemaphoreType.DMA((3,)),
            pltpu.SemaphoreType.REGULAR,
            pltpu.SemaphoreType.REGULAR,
        ],
        compiler_params=pltpu.CompilerParams(
            collective_id=0,
            vmem_limit_bytes=100 * 1024 * 1024,
        ),
    )(x, w_mat)
